# baseline (speedup 1.0000x reference)
import functools

import jax
import jax.numpy as jnp
from jax import lax
from jax.experimental import pallas as pl
from jax.experimental.pallas import tpu as pltpu
from jax.experimental.pallas import tpu_sc as plsc

B, T, D = 16, 4096, 300
TD = T * D
NC, NS = 2, 16
NW = NC * NS
SPAN = TD // NW
SUB = 2400
NSUB = SPAN // SUB
LANES = 16


def _body(x_hbm, xlen_hbm, out_hbm, buf, zbuf, xlen_v):
    w = lax.axis_index("s") * NC + lax.axis_index("c")

    pltpu.sync_copy(xlen_hbm, xlen_v)

    zeros16 = jnp.zeros((LANES,), jnp.float32)

    def zinit(j, carry):
        zbuf[pl.ds(j * LANES, LANES)] = zeros16
        return carry

    lax.fori_loop(0, SPAN // LANES, zinit, 0)

    xlen_vec = xlen_v[...]
    rel_vec = jnp.clip(xlen_vec * D - w * SPAN, 0, SPAN)

    for b in range(B):
        rel = rel_vec[b]
        base = b * TD + w * SPAN

        @pl.when(rel == 0)
        def _zero_span():
            pltpu.sync_copy(zbuf, out_hbm.at[pl.ds(base, SPAN)])

        @pl.when(rel == SPAN)
        def _copy_span():
            pltpu.sync_copy(x_hbm.at[pl.ds(base, SPAN)], buf)
            pltpu.sync_copy(buf, out_hbm.at[pl.ds(base, SPAN)])

        @pl.when((rel > 0) & (rel < SPAN))
        def _straddle_span():
            pltpu.sync_copy(x_hbm.at[pl.ds(base, SPAN)], buf)
            nfull = rel // SUB
            has_part = jnp.where(rel - nfull * SUB > 0, 1, 0)
            nbuf_subs = nfull + has_part

            lo = rel // LANES
            hi = nbuf_subs * (SUB // LANES)

            def mask_body(j, carry):
                vec = buf[pl.ds(j * LANES, LANES)]
                pos = j * LANES + lax.iota(jnp.int32, LANES)
                buf[pl.ds(j * LANES, LANES)] = jnp.where(pos < rel, vec, 0.0)
                return carry

            lax.fori_loop(lo, hi, mask_body, 0)

            def sub_body(s, carry):
                @pl.when(s < nbuf_subs)
                def _from_buf():
                    pltpu.sync_copy(buf.at[pl.ds(s * SUB, SUB)],
                                    out_hbm.at[pl.ds(base + s * SUB, SUB)])

                @pl.when(s >= nbuf_subs)
                def _from_zeros():
                    pltpu.sync_copy(zbuf.at[pl.ds(0, SUB)],
                                    out_hbm.at[pl.ds(base + s * SUB, SUB)])

                return carry

            lax.fori_loop(0, NSUB, sub_body, 0)


def _masked_copy(x_flat, x_len):
    run = pl.kernel(
        _body,
        mesh=plsc.VectorSubcoreMesh(core_axis_name="c", subcore_axis_name="s"),
        out_type=jax.ShapeDtypeStruct((B * TD,), jnp.float32),
        scratch_types=[
            pltpu.VMEM((SPAN,), jnp.float32),
            pltpu.VMEM((SPAN,), jnp.float32),
            pltpu.VMEM((B,), jnp.int32),
        ],
    )
    return run(x_flat, x_len)


def kernel(x, x_len):
    out = _masked_copy(x.reshape(-1), x_len.astype(jnp.int32))
    return out.reshape(B, T, D)


# speedup vs baseline: 1.0448x; 1.0448x over previous
import jax
import jax.numpy as jnp
from jax import lax
from jax.experimental import pallas as pl
from jax.experimental.pallas import tpu as pltpu
from jax.experimental.pallas import tpu_sc as plsc

B, T, D = 16, 4096, 300
TD = T * D
NC, NS = 2, 16
NW = NC * NS
SPAN = TD // NW
SUB = 2400
NSUB = SPAN // SUB
LANES = 16


def _body(x_hbm, xlen_hbm, out_hbm, buf0, buf1, zbuf, xlen_v,
          in_sem0, in_sem1, out_sem0, out_sem1):
    w = lax.axis_index("s") * NC + lax.axis_index("c")
    bufs = (buf0, buf1)
    in_sems = (in_sem0, in_sem1)
    out_sems = (out_sem0, out_sem1)

    pltpu.sync_copy(xlen_hbm, xlen_v)

    zeros16 = jnp.zeros((LANES,), jnp.float32)

    def zinit(j, carry):
        zbuf[pl.ds(j * LANES, LANES)] = zeros16
        return carry

    lax.fori_loop(0, SPAN // LANES, zinit, 0)

    xlen_vec = xlen_v[...]
    rel_vec = jnp.clip(xlen_vec * D - w * SPAN, 0, SPAN)

    for b in range(B):
        s = b % 2
        rel = rel_vec[b]
        base = b * TD + w * SPAN
        out_span = out_hbm.at[pl.ds(base, SPAN)]

        if b >= 2:
            pltpu.make_async_copy(bufs[s], out_span, out_sems[s]).wait()

        @pl.when(rel == 0)
        def _zero_span():
            pltpu.async_copy(zbuf, out_span, out_sems[s])

        @pl.when(rel > 0)
        def _load_span():
            pltpu.async_copy(x_hbm.at[pl.ds(base, SPAN)], bufs[s], in_sems[s])
            pltpu.make_async_copy(
                x_hbm.at[pl.ds(base, SPAN)], bufs[s], in_sems[s]).wait()

        @pl.when(rel == SPAN)
        def _copy_span():
            pltpu.async_copy(bufs[s], out_span, out_sems[s])

        @pl.when((rel > 0) & (rel < SPAN))
        def _straddle_span():
            nfull = rel // SUB
            has_part = jnp.where(rel - nfull * SUB > 0, 1, 0)
            nbuf_subs = nfull + has_part

            lo = rel // LANES
            hi = nbuf_subs * (SUB // LANES)

            def mask_body(j, carry):
                vec = bufs[s][pl.ds(j * LANES, LANES)]
                pos = j * LANES + lax.iota(jnp.int32, LANES)
                bufs[s][pl.ds(j * LANES, LANES)] = jnp.where(pos < rel, vec, 0.0)
                return carry

            lax.fori_loop(lo, hi, mask_body, 0)

            def sub_body(k, carry):
                dst = out_hbm.at[pl.ds(base + k * SUB, SUB)]

                @pl.when(k < nbuf_subs)
                def _from_buf():
                    pltpu.async_copy(bufs[s].at[pl.ds(k * SUB, SUB)], dst,
                                     out_sems[s])

                @pl.when(k >= nbuf_subs)
                def _from_zeros():
                    pltpu.async_copy(zbuf.at[pl.ds(0, SUB)], dst, out_sems[s])

                return carry

            lax.fori_loop(0, NSUB, sub_body, 0)

    for s in range(2):
        b = B - 2 + s
        base = b * TD + w * SPAN
        pltpu.make_async_copy(
            bufs[s], out_hbm.at[pl.ds(base, SPAN)], out_sems[s]).wait()


def _masked_copy(x_flat, x_len):
    run = pl.kernel(
        _body,
        mesh=plsc.VectorSubcoreMesh(core_axis_name="c", subcore_axis_name="s"),
        out_type=jax.ShapeDtypeStruct((B * TD,), jnp.float32),
        scratch_types=[
            pltpu.VMEM((SPAN,), jnp.float32),
            pltpu.VMEM((SPAN,), jnp.float32),
            pltpu.VMEM((SPAN,), jnp.float32),
            pltpu.VMEM((B,), jnp.int32),
            pltpu.SemaphoreType.DMA,
            pltpu.SemaphoreType.DMA,
            pltpu.SemaphoreType.DMA,
            pltpu.SemaphoreType.DMA,
        ],
    )
    return run(x_flat, x_len)


def kernel(x, x_len):
    out = _masked_copy(x.reshape(-1), x_len.astype(jnp.int32))
    return out.reshape(B, T, D)


# speedup vs baseline: 1.8246x; 1.7464x over previous
import jax
import jax.numpy as jnp
from jax import lax
from jax.experimental import pallas as pl
from jax.experimental.pallas import tpu as pltpu

B, T, D = 16, 4096, 300
BT = 512


def _tc_body(xlen_ref, x_ref, o_ref):
    b = pl.program_id(0)
    tb = pl.program_id(1)
    xlen = xlen_ref[b]
    rows = tb * BT + lax.broadcasted_iota(jnp.int32, (1, BT, 1), 1)
    o_ref[...] = jnp.where(rows < xlen, x_ref[...], 0.0)


def _masked_copy_tc(x, x_len):
    grid_spec = pltpu.PrefetchScalarGridSpec(
        num_scalar_prefetch=1,
        grid=(B, T // BT),
        in_specs=[
            pl.BlockSpec(
                (1, BT, D),
                lambda b, tb, xlen: (b, jnp.minimum(tb, (xlen[b] - 1) // BT), 0),
            ),
        ],
        out_specs=pl.BlockSpec((1, BT, D), lambda b, tb, xlen: (b, tb, 0)),
    )
    return pl.pallas_call(
        _tc_body,
        grid_spec=grid_spec,
        out_shape=jax.ShapeDtypeStruct((B, T, D), jnp.float32),
    )(x_len, x)


def kernel(x, x_len):
    return _masked_copy_tc(x, x_len.astype(jnp.int32))
